# baseline (speedup 1.0000x reference)
import functools

import jax
import jax.numpy as jnp
from jax import lax
from jax.experimental import pallas as pl
from jax.experimental.pallas import tpu as pltpu
from jax.experimental.pallas import tpu_sc as plsc

NEG_SLOPE = (1.0 / 8.0 + 1.0 / 3.0) / 2.0

N, E, D = 10000, 320000, 128
NC, NS = 2, 16
SUB = 128
K_REAL = E // SUB
K1 = 160
K2 = 80
G = 16
NG1 = K1 // G
NG2 = K2 // G
EPAD = NS * K1 * SUB
NPAD = 10240
RPT = NPAD // NS

_mesh = plsc.VectorSubcoreMesh(core_axis_name="c", subcore_axis_name="s")


def _zero_rows(rows_ref):
    def outer(r, _):
        def inner(c, _):
            rows_ref[r, pl.ds(c * 16, 16)] = jnp.zeros((16,), jnp.float32)
            return 0
        return lax.fori_loop(0, D // 16, inner, 0)
    lax.fori_loop(0, SUB, outer, 0)


def _zero_vec(vec_ref, n):
    def body(i, _):
        vec_ref[pl.ds(i * 16, 16)] = jnp.zeros((16,), jnp.float32)
        return 0
    lax.fori_loop(0, n // 16, body, 0)


def _fill_ones(vec_ref, n):
    def body(i, _):
        vec_ref[pl.ds(i * 16, 16)] = jnp.ones((16,), jnp.float32)
        return 0
    lax.fori_loop(0, n // 16, body, 0)


def _wipe_acc(zrows, acc, base):
    for r in range(RPT // SUB):
        pltpu.sync_copy(zrows, acc.at[pl.ds(base + r * SUB, SUB)])


def _sweep(ng, load_idx, fetch, do_acc):
    def group(g, _):
        load_idx(g)
        for j in range(G):
            fetch(g, j)
            do_acc(j)
        return 0

    lax.fori_loop(0, ng, group, 0)


@functools.partial(
    pl.kernel,
    out_type=[
        jax.ShapeDtypeStruct((NPAD, D), jnp.float32),
        jax.ShapeDtypeStruct((NPAD,), jnp.float32),
        jax.ShapeDtypeStruct((NPAD, D), jnp.float32),
    ],
    mesh=_mesh,
    scratch_types=[
        pltpu.VMEM((G, SUB), jnp.int32),
        pltpu.VMEM((G, SUB), jnp.int32),
        pltpu.VMEM((SUB, D), jnp.float32),
        pltpu.VMEM((SUB,), jnp.float32),
        pltpu.VMEM((RPT,), jnp.float32),
        pltpu.VMEM_SHARED((NPAD, D), jnp.float32),
        pltpu.VMEM_SHARED((NPAD,), jnp.float32),
    ],
)
def _sc_pass1(src_hbm, dst_hbm, ef_hbm, x_hbm, b_out, deg_out, a0_out,
              srcv, dstv, rows, ones, zvec, acc, dacc):
    cid = lax.axis_index("c")
    sid = lax.axis_index("s")
    base = sid * RPT

    _zero_rows(rows)
    _wipe_acc(rows, acc, base)

    @pl.when(cid == 0)
    def _():
        _fill_ones(ones, SUB)
        _zero_vec(zvec, RPT)
        pltpu.sync_copy(zvec, dacc.at[pl.ds(base, RPT)])

    plsc.subcore_barrier()

    @pl.when(cid == 0)
    def _():
        def load_idx(g):
            pltpu.sync_copy(dst_hbm.at[pl.ds(sid * K1 + g * G, G)], dstv)

        def fetch(g, j):
            kg = sid * K1 + g * G + j
            kg = jnp.where(kg < K_REAL, kg, 0)
            pltpu.sync_copy(ef_hbm.at[pl.ds(kg * SUB, SUB)], rows)

        def do_acc(j):
            pltpu.sync_copy(rows, acc.at[dstv.at[j]], add=True)
            pltpu.sync_copy(ones, dacc.at[dstv.at[j]], add=True)

        _sweep(NG1, load_idx, fetch, do_acc)

    @pl.when(cid == 1)
    def _():
        def load_idx(g):
            pltpu.sync_copy(src_hbm.at[pl.ds(sid * K1 + g * G, G)], srcv)
            pltpu.sync_copy(dst_hbm.at[pl.ds(sid * K1 + g * G, G)], dstv)

        def fetch(g, j):
            pltpu.sync_copy(x_hbm.at[srcv.at[j]], rows)

        def do_acc(j):
            pltpu.sync_copy(rows, acc.at[dstv.at[j]], add=True)

        _sweep(NG1, load_idx, fetch, do_acc)

    plsc.subcore_barrier()

    @pl.when(cid == 0)
    def _():
        pltpu.sync_copy(acc.at[pl.ds(base, RPT)], b_out.at[pl.ds(base, RPT)])
        pltpu.sync_copy(dacc.at[pl.ds(base, RPT)], deg_out.at[pl.ds(base, RPT)])

    @pl.when(cid == 1)
    def _():
        pltpu.sync_copy(acc.at[pl.ds(base, RPT)], a0_out.at[pl.ds(base, RPT)])


@functools.partial(
    pl.kernel,
    out_type=[
        jax.ShapeDtypeStruct((NPAD, D), jnp.float32),
        jax.ShapeDtypeStruct((NPAD, D), jnp.float32),
    ],
    mesh=_mesh,
    scratch_types=[
        pltpu.VMEM((G, SUB), jnp.int32),
        pltpu.VMEM((G, SUB), jnp.int32),
        pltpu.VMEM((SUB, D), jnp.float32),
        pltpu.VMEM_SHARED((NPAD, D), jnp.float32),
    ],
)
def _sc_pass2(src_hbm, dst_hbm, h_hbm, b_hbm, p0_out, p1_out,
              srcv, dstv, rows, acc):
    cid = lax.axis_index("c")
    sid = lax.axis_index("s")
    wid = cid * NS + sid
    base = sid * RPT

    @pl.when(cid == 0)
    def _():
        pltpu.sync_copy(b_hbm.at[pl.ds(base, RPT)], acc.at[pl.ds(base, RPT)])

    @pl.when(cid == 1)
    def _():
        _zero_rows(rows)
        _wipe_acc(rows, acc, base)

    plsc.subcore_barrier()

    def load_idx(g):
        pltpu.sync_copy(src_hbm.at[pl.ds(wid * K2 + g * G, G)], srcv)
        pltpu.sync_copy(dst_hbm.at[pl.ds(wid * K2 + g * G, G)], dstv)

    def fetch(g, j):
        pltpu.sync_copy(h_hbm.at[srcv.at[j]], rows)

    def do_acc(j):
        pltpu.sync_copy(rows, acc.at[dstv.at[j]], add=True)

    _sweep(NG2, load_idx, fetch, do_acc)

    plsc.subcore_barrier()

    @pl.when(cid == 0)
    def _():
        pltpu.sync_copy(acc.at[pl.ds(base, RPT)], p0_out.at[pl.ds(base, RPT)])

    @pl.when(cid == 1)
    def _():
        pltpu.sync_copy(acc.at[pl.ds(base, RPT)], p1_out.at[pl.ds(base, RPT)])


def _tc_body(x_ref, p_ref, q_ref, inv_ref, iso_ref, w1_ref, w2_ref, w3_ref, o_ref):
    dn = (((1,), (1,)), ((), ()))
    s = (p_ref[...] + q_ref[...]) * inv_ref[...]
    neigh = lax.dot_general(s, w1_ref[...], dn, preferred_element_type=jnp.float32)
    x = x_ref[...]
    s2 = lax.dot_general(x, w2_ref[...], dn, preferred_element_type=jnp.float32)
    s3 = lax.dot_general(x, w3_ref[...], dn, preferred_element_type=jnp.float32)
    h = neigh + s2 + iso_ref[...] * (s3 - s2)
    o_ref[...] = jnp.where(h >= 0, h, h * NEG_SLOPE)


_TC_ROWS = 512


def _tc_layer(x, p, q, invb, isob, w1, w2, w3):
    row_spec = pl.BlockSpec((_TC_ROWS, D), lambda i: (i, 0))
    w_spec = pl.BlockSpec((D, D), lambda i: (0, 0))
    return pl.pallas_call(
        _tc_body,
        grid=(NPAD // _TC_ROWS,),
        in_specs=[row_spec, row_spec, row_spec, row_spec, row_spec,
                  w_spec, w_spec, w_spec],
        out_specs=row_spec,
        out_shape=jax.ShapeDtypeStruct((NPAD, D), jnp.float32),
    )(x, p, q, invb, isob, w1, w2, w3)


def kernel(node_feats, edge_feats, edge_index, W1_0, W2_0, W3_0, W1_1, W2_1, W3_1):
    src = edge_index[0]
    dst = edge_index[1]
    src2d = jnp.pad(src, (0, EPAD - E), constant_values=N).reshape(-1, SUB)
    dst2d = jnp.pad(dst, (0, EPAD - E), constant_values=N).reshape(-1, SUB)
    x_pad = jnp.pad(node_feats, ((0, NPAD - N), (0, 0)))

    b_agg, deg, a0 = _sc_pass1(src2d, dst2d, edge_feats, x_pad)

    inv = 1.0 / jnp.maximum(deg, 1.0)
    iso = (deg == 0.0).astype(jnp.float32)
    invb = jnp.broadcast_to(inv[:, None], (NPAD, D))
    isob = jnp.broadcast_to(iso[:, None], (NPAD, D))

    h1 = _tc_layer(x_pad, a0, b_agg, invb, isob, W1_0, W2_0, W3_0)
    p0, p1 = _sc_pass2(src2d, dst2d, h1, b_agg)
    h2 = _tc_layer(h1, p0, p1, invb, isob, W1_1, W2_1, W3_1)
    return h2[:N]


# speedup vs baseline: 9.7701x; 9.7701x over previous
import functools

import jax
import jax.numpy as jnp
from jax import lax
from jax.experimental import pallas as pl
from jax.experimental.pallas import tpu as pltpu
from jax.experimental.pallas import tpu_sc as plsc

NEG_SLOPE = (1.0 / 8.0 + 1.0 / 3.0) / 2.0

N, E, D = 10000, 320000, 128
NC, NS = 2, 16
SUB = 128
K_REAL = E // SUB
K1 = 160
K2 = 80
G = 16
NG1 = K1 // G
NG2 = K2 // G
EPAD = NS * K1 * SUB
NPAD = 10240
RPT = NPAD // NS

_mesh = plsc.VectorSubcoreMesh(core_axis_name="c", subcore_axis_name="s")


def _zero_rows(rows_ref):
    def outer(r, _):
        def inner(c, _):
            rows_ref[r, pl.ds(c * 16, 16)] = jnp.zeros((16,), jnp.float32)
            return 0
        return lax.fori_loop(0, D // 16, inner, 0)
    lax.fori_loop(0, SUB, outer, 0)


def _zero_vec(vec_ref, n):
    def body(i, _):
        vec_ref[pl.ds(i * 16, 16)] = jnp.zeros((16,), jnp.float32)
        return 0
    lax.fori_loop(0, n // 16, body, 0)


def _fill_ones(vec_ref, n):
    def body(i, _):
        vec_ref[pl.ds(i * 16, 16)] = jnp.ones((16,), jnp.float32)
        return 0
    lax.fori_loop(0, n // 16, body, 0)


def _wipe_acc(zrows, acc, base):
    for r in range(RPT // SUB):
        pltpu.sync_copy(zrows, acc.at[pl.ds(base + r * SUB, SUB)])


def _sweep(ng, load_idx, fetch, do_acc):
    def group(g, _):
        load_idx(g)

        @functools.partial(plsc.parallel_loop, 0, G, unroll=2)
        def _(j):
            o = (j % 2) * SUB
            fetch(g, j, o)
            do_acc(j, o)

        return 0

    lax.fori_loop(0, ng, group, 0)


@functools.partial(
    pl.kernel,
    out_type=[
        jax.ShapeDtypeStruct((NPAD, D), jnp.float32),
        jax.ShapeDtypeStruct((NPAD,), jnp.float32),
        jax.ShapeDtypeStruct((NPAD, D), jnp.float32),
    ],
    mesh=_mesh,
    scratch_types=[
        pltpu.VMEM((G, SUB), jnp.int32),
        pltpu.VMEM((G, SUB), jnp.int32),
        pltpu.VMEM((2 * SUB, D), jnp.float32),
        pltpu.VMEM((SUB,), jnp.float32),
        pltpu.VMEM((RPT,), jnp.float32),
        pltpu.VMEM_SHARED((NPAD, D), jnp.float32),
        pltpu.VMEM_SHARED((NPAD,), jnp.float32),
    ],
)
def _sc_pass1(src_hbm, dst_hbm, ef_hbm, x_hbm, b_out, deg_out, a0_out,
              srcv, dstv, rows, ones, zvec, acc, dacc):
    cid = lax.axis_index("c")
    sid = lax.axis_index("s")
    base = sid * RPT

    _zero_rows(rows)
    _wipe_acc(rows.at[pl.ds(0, SUB)], acc, base)

    @pl.when(cid == 0)
    def _():
        _fill_ones(ones, SUB)
        _zero_vec(zvec, RPT)
        pltpu.sync_copy(zvec, dacc.at[pl.ds(base, RPT)])

    plsc.subcore_barrier()

    @pl.when(cid == 0)
    def _():
        def load_idx(g):
            pltpu.sync_copy(dst_hbm.at[pl.ds(sid * K1 + g * G, G)], dstv)

        def fetch(g, j, o):
            kg = sid * K1 + g * G + j
            kg = jnp.where(kg < K_REAL, kg, 0)
            pltpu.sync_copy(ef_hbm.at[pl.ds(kg * SUB, SUB)],
                            rows.at[pl.ds(o, SUB)])

        def do_acc(j, o):
            pltpu.sync_copy(rows.at[pl.ds(o, SUB)], acc.at[dstv.at[j]],
                            add=True)
            pltpu.sync_copy(ones, dacc.at[dstv.at[j]], add=True)

        _sweep(NG1, load_idx, fetch, do_acc)

    @pl.when(cid == 1)
    def _():
        def load_idx(g):
            pltpu.sync_copy(src_hbm.at[pl.ds(sid * K1 + g * G, G)], srcv)
            pltpu.sync_copy(dst_hbm.at[pl.ds(sid * K1 + g * G, G)], dstv)

        def fetch(g, j, o):
            pltpu.sync_copy(x_hbm.at[srcv.at[j]], rows.at[pl.ds(o, SUB)])

        def do_acc(j, o):
            pltpu.sync_copy(rows.at[pl.ds(o, SUB)], acc.at[dstv.at[j]],
                            add=True)

        _sweep(NG1, load_idx, fetch, do_acc)

    plsc.subcore_barrier()

    @pl.when(cid == 0)
    def _():
        pltpu.sync_copy(acc.at[pl.ds(base, RPT)], b_out.at[pl.ds(base, RPT)])
        pltpu.sync_copy(dacc.at[pl.ds(base, RPT)], deg_out.at[pl.ds(base, RPT)])

    @pl.when(cid == 1)
    def _():
        pltpu.sync_copy(acc.at[pl.ds(base, RPT)], a0_out.at[pl.ds(base, RPT)])


@functools.partial(
    pl.kernel,
    out_type=[
        jax.ShapeDtypeStruct((NPAD, D), jnp.float32),
        jax.ShapeDtypeStruct((NPAD, D), jnp.float32),
    ],
    mesh=_mesh,
    scratch_types=[
        pltpu.VMEM((G, SUB), jnp.int32),
        pltpu.VMEM((G, SUB), jnp.int32),
        pltpu.VMEM((2 * SUB, D), jnp.float32),
        pltpu.VMEM_SHARED((NPAD, D), jnp.float32),
    ],
)
def _sc_pass2(src_hbm, dst_hbm, h_hbm, b_hbm, p0_out, p1_out,
              srcv, dstv, rows, acc):
    cid = lax.axis_index("c")
    sid = lax.axis_index("s")
    wid = cid * NS + sid
    base = sid * RPT

    @pl.when(cid == 0)
    def _():
        pltpu.sync_copy(b_hbm.at[pl.ds(base, RPT)], acc.at[pl.ds(base, RPT)])

    @pl.when(cid == 1)
    def _():
        _zero_rows(rows)
        _wipe_acc(rows.at[pl.ds(0, SUB)], acc, base)

    plsc.subcore_barrier()

    def load_idx(g):
        pltpu.sync_copy(src_hbm.at[pl.ds(wid * K2 + g * G, G)], srcv)
        pltpu.sync_copy(dst_hbm.at[pl.ds(wid * K2 + g * G, G)], dstv)

    def fetch(g, j, o):
        pltpu.sync_copy(h_hbm.at[srcv.at[j]], rows.at[pl.ds(o, SUB)])

    def do_acc(j, o):
        pltpu.sync_copy(rows.at[pl.ds(o, SUB)], acc.at[dstv.at[j]],
                        add=True)

    _sweep(NG2, load_idx, fetch, do_acc)

    plsc.subcore_barrier()

    @pl.when(cid == 0)
    def _():
        pltpu.sync_copy(acc.at[pl.ds(base, RPT)], p0_out.at[pl.ds(base, RPT)])

    @pl.when(cid == 1)
    def _():
        pltpu.sync_copy(acc.at[pl.ds(base, RPT)], p1_out.at[pl.ds(base, RPT)])


def _tc_body(x_ref, p_ref, q_ref, inv_ref, iso_ref, w1_ref, w2_ref, w3_ref, o_ref):
    dn = (((1,), (1,)), ((), ()))
    s = (p_ref[...] + q_ref[...]) * inv_ref[...]
    neigh = lax.dot_general(s, w1_ref[...], dn, preferred_element_type=jnp.float32)
    x = x_ref[...]
    s2 = lax.dot_general(x, w2_ref[...], dn, preferred_element_type=jnp.float32)
    s3 = lax.dot_general(x, w3_ref[...], dn, preferred_element_type=jnp.float32)
    h = neigh + s2 + iso_ref[...] * (s3 - s2)
    o_ref[...] = jnp.where(h >= 0, h, h * NEG_SLOPE)


_TC_ROWS = 512


def _tc_layer(x, p, q, invb, isob, w1, w2, w3):
    row_spec = pl.BlockSpec((_TC_ROWS, D), lambda i: (i, 0))
    w_spec = pl.BlockSpec((D, D), lambda i: (0, 0))
    return pl.pallas_call(
        _tc_body,
        grid=(NPAD // _TC_ROWS,),
        in_specs=[row_spec, row_spec, row_spec, row_spec, row_spec,
                  w_spec, w_spec, w_spec],
        out_specs=row_spec,
        out_shape=jax.ShapeDtypeStruct((NPAD, D), jnp.float32),
    )(x, p, q, invb, isob, w1, w2, w3)


def kernel(node_feats, edge_feats, edge_index, W1_0, W2_0, W3_0, W1_1, W2_1, W3_1):
    src = edge_index[0]
    dst = edge_index[1]
    src2d = jnp.pad(src, (0, EPAD - E), constant_values=N).reshape(-1, SUB)
    dst2d = jnp.pad(dst, (0, EPAD - E), constant_values=N).reshape(-1, SUB)
    x_pad = jnp.pad(node_feats, ((0, NPAD - N), (0, 0)))

    b_agg, deg, a0 = _sc_pass1(src2d, dst2d, edge_feats, x_pad)

    inv = 1.0 / jnp.maximum(deg, 1.0)
    iso = (deg == 0.0).astype(jnp.float32)
    invb = jnp.broadcast_to(inv[:, None], (NPAD, D))
    isob = jnp.broadcast_to(iso[:, None], (NPAD, D))

    h1 = _tc_layer(x_pad, a0, b_agg, invb, isob, W1_0, W2_0, W3_0)
    p0, p1 = _sc_pass2(src2d, dst2d, h1, b_agg)
    h2 = _tc_layer(h1, p0, p1, invb, isob, W1_1, W2_1, W3_1)
    return h2[:N]
